# Initial kernel scaffold; baseline (speedup 1.0000x reference)
import jax
import jax.numpy as jnp
from jax.experimental import pallas as pl


def kernel(input_1, input_2, mask_1, mask_2, word_inds, vocab, Wq, Wk, Wv):
    raise NotImplementedError("write your pallas kernel here")


# speedup vs baseline: 1.8835x; 1.8835x over previous
import functools

import jax
import jax.numpy as jnp
from jax import lax
from jax.experimental import pallas as pl
from jax.experimental.pallas import tpu as pltpu
from jax.experimental.pallas import tpu_sc as plsc

_NC = 2
_NS = 16
_NW = _NC * _NS


def _attn_body(x1_ref, x2_ref, m1_ref, m2_ref, wq_ref, wk_ref, attn_ref):
  x1 = x1_ref[...]
  wq = wq_ref[...]
  wk = wk_ref[...]
  d_k = float(wq.shape[0])
  q = lax.dot_general(x1, wq, (((1,), (1,)), ((), ())),
                      preferred_element_type=jnp.float32)
  qk = lax.dot_general(q, wk, (((1,), (0,)), ((), ())),
                       preferred_element_type=jnp.float32)
  x2 = x2_ref[...]
  scores = jnp.sum(x2 * qk[:, None, :], axis=-1) * (1.0 / (d_k ** 0.5))
  sm = m1_ref[...][:, None] * m2_ref[...]
  scores = jnp.where(sm > 0, scores, jnp.float32(-1e9))
  mx = jnp.max(scores, axis=1, keepdims=True)
  e = jnp.exp(scores - mx)
  s = jnp.sum(e, axis=1, keepdims=True)
  attn_ref[...] = (e / s) * sm


def _attn_weights(x1, x2, m1, m2, wq, wk):
  B, L2, H = x2.shape
  Q = x1.shape[1]
  TB = 128
  grid = (B // TB,)
  return pl.pallas_call(
      _attn_body,
      grid=grid,
      in_specs=[
          pl.BlockSpec((TB, Q), lambda i: (i, 0)),
          pl.BlockSpec((TB, L2, H), lambda i: (i, 0, 0)),
          pl.BlockSpec((TB,), lambda i: (i,)),
          pl.BlockSpec((TB, L2), lambda i: (i, 0)),
          pl.BlockSpec((H, Q), lambda i: (0, 0)),
          pl.BlockSpec((H, H), lambda i: (0, 0)),
      ],
      out_specs=pl.BlockSpec((TB, L2), lambda i: (i, 0)),
      out_shape=jax.ShapeDtypeStruct((B, L2), jnp.float32),
  )(x1, x2, m1, m2, wq, wk)


def _make_sc_scatter(B, LP, V):
  rows_per_worker = B // _NW
  n_chunks = LP // 16
  CT = -(-V // 128)
  mesh = plsc.VectorSubcoreMesh(core_axis_name="c", subcore_axis_name="s",
                                num_cores=_NC)

  @functools.partial(
      pl.kernel,
      out_type=jax.ShapeDtypeStruct((B // 8, CT, 8, 128), jnp.float32),
      mesh=mesh,
      compiler_params=pltpu.CompilerParams(needs_layout_passes=False),
      scratch_types=[
          pltpu.VMEM((LP,), jnp.int32),
          pltpu.VMEM((LP,), jnp.float32),
          pltpu.VMEM((CT, 128), jnp.float32),
      ],
  )
  def sc_scatter(widx_hbm, attw_hbm, out_hbm, wbuf, vbuf, slab):
    cid = lax.axis_index("c")
    sid = lax.axis_index("s")
    wid = sid * _NC + cid
    row0 = wid * rows_per_worker

    z16 = jnp.zeros((16,), jnp.float32)

    def slab_zero(i, carry):
      for j in range(8):
        slab[i, pl.ds(j * 16, 16)] = z16
      return carry
    lax.fori_loop(0, CT, slab_zero, 0)

    lane = lax.broadcasted_iota(jnp.int32, (16,), 0)
    lane_masks = [lane == k for k in range(16)]

    def row_body(r, carry):
      b = row0 + r
      pltpu.sync_copy(widx_hbm.at[pl.ds(b * LP, LP)], wbuf)
      pltpu.sync_copy(attw_hbm.at[pl.ds(b * LP, LP)], vbuf)
      for c in range(n_chunks):
        w = wbuf[pl.ds(c * 16, 16)]
        v = vbuf[pl.ds(c * 16, 16)]
        wt = lax.shift_right_logical(w, 7)
        wl = lax.bitwise_and(w, 127)
        for k in range(16):
          plsc.addupdate_scatter(slab, [wt, wl], v, mask=lane_masks[k])
      pltpu.sync_copy(slab, out_hbm.at[b // 8, :, b % 8, :])
      for c in range(n_chunks):
        w = wbuf[pl.ds(c * 16, 16)]
        wt = lax.shift_right_logical(w, 7)
        wl = lax.bitwise_and(w, 127)
        for k in range(16):
          plsc.store_scatter(slab, [wt, wl], z16, mask=lane_masks[k])
      return carry

    lax.fori_loop(0, rows_per_worker, row_body, 0)

  return sc_scatter


def kernel(input_1, input_2, mask_1, mask_2, word_inds, vocab, Wq, Wk, Wv):
  del Wv
  B, L1, Q = input_1.shape
  L2 = input_2.shape[1]
  V = 100000

  x1 = input_1.reshape(B, Q).astype(jnp.float32)
  m1 = mask_1.reshape(B * L1)[:B].astype(jnp.float32)
  attn = _attn_weights(x1, input_2.astype(jnp.float32), m1,
                       mask_2.astype(jnp.float32), Wq, Wk)

  LP = 256
  idx = (word_inds - 1 + (vocab - V)).astype(jnp.int32)
  idx_p = jnp.pad(idx, ((0, 0), (0, LP - L2))).reshape(B * LP)
  att_p = jnp.pad(attn, ((0, 0), (0, LP - L2))).reshape(B * LP)
  CT = -(-V // 128)
  t4 = _make_sc_scatter(B, LP, V)(idx_p, att_p)
  dist = t4.transpose(0, 2, 1, 3).reshape(B, CT * 128)[:, :V]
  attn_heads = attn.reshape(B, 1, L1, L2)
  return (dist, attn_heads)
